# baseline (speedup 1.0000x reference)
import functools

import jax
import jax.numpy as jnp
from jax import lax
from jax.experimental import pallas as pl
from jax.experimental.pallas import tpu as pltpu
from jax.experimental.pallas import tpu_sc as plsc


def kernel(x):
    mesh = plsc.VectorSubcoreMesh(core_axis_name="c", subcore_axis_name="s")

    @functools.partial(
        pl.kernel,
        mesh=mesh,
        out_type=jax.ShapeDtypeStruct((3, 128), x.dtype),
    )
    def gather_rows(x_hbm, out_hbm):
        wid = lax.axis_index("s") * 2 + lax.axis_index("c")

        @pl.when(wid == 0)
        def _():
            pltpu.sync_copy(x_hbm.at[pl.ds(2, 1)], out_hbm.at[pl.ds(0, 1)])
            pltpu.sync_copy(x_hbm.at[pl.ds(4, 2)], out_hbm.at[pl.ds(1, 2)])

    return gather_rows(x)


# speedup vs baseline: 1.1791x; 1.1791x over previous
import functools

import jax
import jax.numpy as jnp
from jax import lax
from jax.experimental import pallas as pl
from jax.experimental.pallas import tpu as pltpu
from jax.experimental.pallas import tpu_sc as plsc


def kernel(x):
    mesh = plsc.ScalarSubcoreMesh(axis_name="c", num_cores=1)

    @functools.partial(
        pl.kernel,
        mesh=mesh,
        out_type=jax.ShapeDtypeStruct((3, 128), x.dtype),
    )
    def gather_rows(x_hbm, out_hbm):
        pltpu.sync_copy(x_hbm.at[pl.ds(2, 1)], out_hbm.at[pl.ds(0, 1)])
        pltpu.sync_copy(x_hbm.at[pl.ds(4, 2)], out_hbm.at[pl.ds(1, 2)])

    return gather_rows(x)


# speedup vs baseline: 18.2224x; 15.4539x over previous
import jax
import jax.numpy as jnp
from jax.experimental import pallas as pl
from jax.experimental.pallas import tpu as pltpu


def _gather_kernel(x_hbm, o_hbm, sem0, sem1):
    c0 = pltpu.make_async_copy(x_hbm.at[pl.ds(2, 1)], o_hbm.at[pl.ds(0, 1)], sem0)
    c1 = pltpu.make_async_copy(x_hbm.at[pl.ds(4, 2)], o_hbm.at[pl.ds(1, 2)], sem1)
    c0.start()
    c1.start()
    c0.wait()
    c1.wait()


def kernel(x):
    return pl.pallas_call(
        _gather_kernel,
        out_shape=jax.ShapeDtypeStruct((3, 128), x.dtype),
        in_specs=[pl.BlockSpec(memory_space=pl.ANY)],
        out_specs=pl.BlockSpec(memory_space=pl.ANY),
        scratch_shapes=[pltpu.SemaphoreType.DMA, pltpu.SemaphoreType.DMA],
    )(x)


# speedup vs baseline: 18.4023x; 1.0099x over previous
import jax
import jax.numpy as jnp
from jax.experimental import pallas as pl
from jax.experimental.pallas import tpu as pltpu


def _gather_kernel(x_hbm, o_hbm, sem):
    c0 = pltpu.make_async_copy(x_hbm.at[pl.ds(2, 1)], o_hbm.at[pl.ds(0, 1)], sem)
    c1 = pltpu.make_async_copy(x_hbm.at[pl.ds(4, 2)], o_hbm.at[pl.ds(1, 2)], sem)
    c0.start()
    c1.start()
    c0.wait()
    c1.wait()


def kernel(x):
    return pl.pallas_call(
        _gather_kernel,
        out_shape=jax.ShapeDtypeStruct((3, 128), x.dtype),
        in_specs=[pl.BlockSpec(memory_space=pl.ANY)],
        out_specs=pl.BlockSpec(memory_space=pl.ANY),
        scratch_shapes=[pltpu.SemaphoreType.DMA],
    )(x)
